# Initial kernel scaffold; baseline (speedup 1.0000x reference)
import jax
import jax.numpy as jnp
from jax.experimental import pallas as pl


def kernel(x, edge_index, edge_attr, W_atom, b_atom, W_bond, b_bond, W_gnn, b_gnn):
    raise NotImplementedError("write your pallas kernel here")


# speedup vs baseline: 4.5179x; 4.5179x over previous
import functools

import jax
import jax.numpy as jnp
from jax import lax
from jax.experimental import pallas as pl
from jax.experimental.pallas import tpu as pltpu
from jax.experimental.pallas import tpu_sc as plsc

N_NODES = 10000
N_EDGES = 320000
EMB = 128
N_LAYERS = 20
RES_EVERY = 2

NW = 32
CHUNK = 128
NCHUNK = 79
EDGES_PER_W = NCHUNK * CHUNK
PAD_E = NW * EDGES_PER_W
PAD_N = 10112
STRIPE = PAD_N // 16

ROW_BLK = 1000

_sc_mesh = plsc.VectorSubcoreMesh(core_axis_name="c", subcore_axis_name="s")


@functools.partial(
    pl.kernel,
    mesh=_sc_mesh,
    out_type=jax.ShapeDtypeStruct((2, PAD_N, EMB), jnp.float32),
    scratch_types=[
        pltpu.VMEM((NCHUNK, CHUNK), jnp.int32),
        pltpu.VMEM((NCHUNK, CHUNK), jnp.int32),
        pltpu.VMEM((CHUNK, EMB), jnp.float32),
        pltpu.VMEM_SHARED((PAD_N, EMB), jnp.float32),
        pltpu.SemaphoreType.DMA,
    ],
)
def _sc_agg(h_hbm, src_hbm, dst_hbm, zeros_hbm, out_hbm,
            sidx, didx, rows, agg, sem):
    c = lax.axis_index("c")
    s = lax.axis_index("s")
    wid = s * 2 + c
    pltpu.sync_copy(zeros_hbm, agg.at[pl.ds(s * STRIPE, STRIPE)])
    pltpu.sync_copy(src_hbm.at[wid], sidx)
    pltpu.sync_copy(dst_hbm.at[wid], didx)
    plsc.subcore_barrier()

    def body(i, carry):
        pltpu.async_copy(h_hbm.at[sidx.at[i]], rows, sem).wait()
        pltpu.sync_copy(rows, agg.at[didx.at[i]], add=True)
        return carry

    lax.fori_loop(0, NCHUNK, body, 0)
    plsc.subcore_barrier()
    pltpu.sync_copy(agg.at[pl.ds(s * STRIPE, STRIPE)],
                    out_hbm.at[c, pl.ds(s * STRIPE, STRIPE)])


def _embed_body(x_ref, w_ref, b_ref, o_ref):
    o_ref[...] = lax.dot_general(
        x_ref[...], w_ref[...], (((1,), (0,)), ((), ())),
        preferred_element_type=jnp.float32,
        precision=lax.Precision.HIGHEST,
    ) + b_ref[0:1, :]


def _tc_embed(xp, wp, b8):
    k = xp.shape[1]
    return pl.pallas_call(
        _embed_body,
        grid=(N_NODES // ROW_BLK,),
        in_specs=[
            pl.BlockSpec((ROW_BLK, k), lambda r: (r, 0)),
            pl.BlockSpec((k, EMB), lambda r: (0, 0)),
            pl.BlockSpec((8, EMB), lambda r: (0, 0)),
        ],
        out_specs=pl.BlockSpec((ROW_BLK, EMB), lambda r: (r, 0)),
        out_shape=jax.ShapeDtypeStruct((N_NODES, EMB), jnp.float32),
    )(xp, wp, b8)


def _upd_body(h_ref, a_ref, w_ref, b_ref, o_ref):
    rst = h_ref[...] + a_ref[0] + a_ref[1]
    y = lax.dot_general(
        rst, w_ref[...], (((1,), (0,)), ((), ())),
        preferred_element_type=jnp.float32,
        precision=lax.Precision.HIGHEST,
    ) + b_ref[0:1, :]
    o_ref[...] = jnp.maximum(y, 0.0)


def _upd_res_body(h_ref, a_ref, w_ref, b_ref, r_ref, o_ref):
    rst = h_ref[...] + a_ref[0] + a_ref[1]
    y = lax.dot_general(
        rst, w_ref[...], (((1,), (0,)), ((), ())),
        preferred_element_type=jnp.float32,
        precision=lax.Precision.HIGHEST,
    ) + b_ref[0:1, :]
    o_ref[...] = jnp.maximum(y, 0.0) + r_ref[...]


def _tc_update(h, agg, w, b8, res):
    specs = [
        pl.BlockSpec((ROW_BLK, EMB), lambda r: (r, 0)),
        pl.BlockSpec((2, ROW_BLK, EMB), lambda r: (0, r, 0)),
        pl.BlockSpec((EMB, EMB), lambda r: (0, 0)),
        pl.BlockSpec((8, EMB), lambda r: (0, 0)),
    ]
    args = [h, agg, w, b8]
    body = _upd_body
    if res is not None:
        specs.append(pl.BlockSpec((ROW_BLK, EMB), lambda r: (r, 0)))
        args.append(res)
        body = _upd_res_body
    return pl.pallas_call(
        body,
        grid=(N_NODES // ROW_BLK,),
        in_specs=specs,
        out_specs=pl.BlockSpec((ROW_BLK, EMB), lambda r: (r, 0)),
        out_shape=jax.ShapeDtypeStruct((N_NODES, EMB), jnp.float32),
    )(*args)


def kernel(x, edge_index, edge_attr, W_atom, b_atom, W_bond, b_bond,
           W_gnn, b_gnn):
    del edge_attr, W_bond, b_bond

    src = edge_index[0].astype(jnp.int32)
    dst = edge_index[1].astype(jnp.int32)
    pad = PAD_E - N_EDGES
    src_p = jnp.concatenate([src, jnp.zeros((pad,), jnp.int32)])
    dst_p = jnp.concatenate([dst, jnp.full((pad,), N_NODES, jnp.int32)])
    src_p = src_p.reshape(NW, NCHUNK, CHUNK)
    dst_p = dst_p.reshape(NW, NCHUNK, CHUNK)
    zeros_stripe = jnp.zeros((STRIPE, EMB), jnp.float32)

    kpad = (-x.shape[1]) % 8
    xp = jnp.pad(x, ((0, 0), (0, kpad)))
    wp = jnp.pad(W_atom, ((0, kpad), (0, 0)))
    b_atom8 = jnp.broadcast_to(b_atom[None, :], (8, EMB))
    h = _tc_embed(xp, wp, b_atom8)

    res = h
    for i in range(N_LAYERS):
        agg = _sc_agg(h, src_p, dst_p, zeros_stripe)
        b8 = jnp.broadcast_to(b_gnn[i][None, :], (8, EMB))
        if (i + 1) % RES_EVERY == 0:
            h = _tc_update(h, agg, W_gnn[i], b8, res)
            res = h
        else:
            h = _tc_update(h, agg, W_gnn[i], b8, None)
    return h
